# baseline (speedup 1.0000x reference)
import functools

import jax
import jax.numpy as jnp
from jax import lax
from jax.experimental import pallas as pl
from jax.experimental.pallas import tpu as pltpu
from jax.experimental.pallas import tpu_sc as plsc

N = 10000
E = 320000
D_IN = 128
H = 64
D_OUT = 128

NC = 2
NS = 16
NW = NC * NS
T = E // NW
B = 80
M = T // B
MC = 25
CH = M // MC
N_PAD = 10240
ROWS_PER_TILE = N_PAD // NS

_mesh = plsc.VectorSubcoreMesh(
    core_axis_name="c", subcore_axis_name="s", num_cores=NC, num_subcores=NS)
_sc_params = pltpu.CompilerParams(use_tc_tiling_on_sc=False, needs_layout_passes=False)


def _worker_id():
    cid = lax.axis_index("c")
    sid = lax.axis_index("s")
    return cid * NS + sid, cid, sid


def _load_edge_block(e3d_hbm, buf, wid):
    pltpu.sync_copy(e3d_hbm.at[wid], buf)


def _fill_zeros(zbuf):
    z = jnp.zeros((16,), jnp.float32)
    if zbuf.ndim == 2:
        for r in range(zbuf.shape[0]):
            for c in range(0, zbuf.shape[1], 16):
                zbuf[r, pl.ds(c, 16)] = z
    else:
        for c in range(0, zbuf.shape[0], 16):
            zbuf[pl.ds(c, 16)] = z


def _zero_stripe_2d(acc, zblk, sid):
    zr = zblk.shape[0]
    base = sid * ROWS_PER_TILE
    for i in range(ROWS_PER_TILE // zr):
        pltpu.sync_copy(zblk, acc.at[pl.ds(base + i * zr, zr)])


def _zero_stripe_1d(acc, zstripe, sid):
    pltpu.sync_copy(zstripe, acc.at[pl.ds(sid * ROWS_PER_TILE, ROWS_PER_TILE)])


def _writeback_2d(acc, out_hbm, cid, sid):
    base = sid * ROWS_PER_TILE
    pltpu.sync_copy(acc.at[pl.ds(base, ROWS_PER_TILE)],
                    out_hbm.at[cid, pl.ds(base, ROWS_PER_TILE)])


def _writeback_1d(acc, out_hbm, cid, sid):
    base = sid * ROWS_PER_TILE
    pltpu.sync_copy(acc.at[pl.ds(base, ROWS_PER_TILE)],
                    out_hbm.at[cid, pl.ds(base, ROWS_PER_TILE)])


@functools.partial(
    pl.kernel,
    out_type=jax.ShapeDtypeStruct((NC, N_PAD), jnp.float32),
    mesh=_mesh,
    compiler_params=_sc_params,
    scratch_types=[
        pltpu.VMEM((M, B), jnp.int32),
        pltpu.VMEM((B,), jnp.float32),
        pltpu.VMEM((ROWS_PER_TILE,), jnp.float32),
        pltpu.VMEM_SHARED((N_PAD,), jnp.float32),
    ],
)
def _sc_degree(e_dst_hbm, out_hbm, dstb, ones, zstripe, acc):
    wid, cid, sid = _worker_id()
    _fill_zeros(zstripe)
    one = jnp.ones((16,), jnp.float32)
    for c in range(0, B, 16):
        ones[pl.ds(c, 16)] = one
    _zero_stripe_1d(acc, zstripe, sid)
    _load_edge_block(e_dst_hbm, dstb, wid)
    plsc.subcore_barrier()

    def body(j, carry):
        pltpu.sync_copy(ones, acc.at[dstb.at[j]], add=True)
        return carry

    lax.fori_loop(0, M, body, None)
    plsc.subcore_barrier()
    _writeback_1d(acc, out_hbm, cid, sid)


def _make_sc_gather_scatter(D, stage_table=False):
    NB = 4

    scratch = [
        pltpu.VMEM((MC, B), jnp.int32),
        pltpu.VMEM((MC, B), jnp.int32),
        pltpu.VMEM((B, D), jnp.float32),
        pltpu.VMEM((B, D), jnp.float32),
        pltpu.VMEM((B, D), jnp.float32),
        pltpu.VMEM((B, D), jnp.float32),
        pltpu.VMEM_SHARED((N_PAD, D), jnp.float32),
        [pltpu.SemaphoreType.DMA] * 4,
        [pltpu.SemaphoreType.DMA] * 4,
    ]
    if stage_table:
        scratch.append(pltpu.VMEM_SHARED((N, D), jnp.float32))

    @functools.partial(
        pl.kernel,
        out_type=jax.ShapeDtypeStruct((NC, N_PAD, D), jnp.float32),
        mesh=_mesh,
        compiler_params=_sc_params,
        scratch_types=scratch,
    )
    def _sc_gs(e_src_hbm, e_dst_hbm, table_hbm, out_hbm,
               srcb, dstb, b0, b1, b2, b3, acc, gsems, ssems, *maybe_tbl):
        wid, cid, sid = _worker_id()
        bufs = (b0, b1, b2, b3)
        _fill_zeros(b0)
        _zero_stripe_2d(acc, b0, sid)
        if stage_table:
            tbl = maybe_tbl[0]
            rows = N // NS
            pltpu.sync_copy(table_hbm.at[pl.ds(sid * rows, rows)],
                            tbl.at[pl.ds(sid * rows, rows)])
            table_src = tbl
        else:
            table_src = table_hbm
        plsc.subcore_barrier()

        def gstart(j, p):
            pltpu.async_copy(table_src.at[srcb.at[j]], bufs[p], gsems[p])

        def gwait(p):
            pltpu.make_async_copy(table_src.at[srcb.at[0]], bufs[p], gsems[p]).wait()

        def sstart(j, p):
            pltpu.async_copy(bufs[p], acc.at[dstb.at[j]], ssems[p], add=True)

        def swait(p):
            pltpu.make_async_copy(bufs[p], acc.at[dstb.at[0]], ssems[p]).wait()

        def chunk(c, carry):
            pltpu.sync_copy(e_src_hbm.at[wid, pl.ds(c * MC, MC)], srcb)
            pltpu.sync_copy(e_dst_hbm.at[wid, pl.ds(c * MC, MC)], dstb)
            gstart(0, 0)
            gstart(1, 1)

            def body(i, carry2):
                j0 = 4 * i
                for p in range(NB):
                    j = j0 + p

                    @pl.when(j < MC)
                    def _(j=j, p=p):
                        pn = (p + 2) % NB

                        @pl.when(j + 2 < MC)
                        def _():
                            @pl.when(j >= 2)
                            def _():
                                swait(pn)
                            gstart(j + 2, pn)

                        gwait(p)
                        sstart(j, p)
                return carry2

            lax.fori_loop(0, (MC + NB - 1) // NB, body, None)
            for p in range(NB):
                swait(p)
            return carry

        lax.fori_loop(0, CH, chunk, None)
        plsc.subcore_barrier()
        _writeback_2d(acc, out_hbm, cid, sid)

    return _sc_gs


_sc_gcn_gather_scatter_64 = _make_sc_gather_scatter(64, stage_table=True)
_sc_gcn_gather_scatter_128 = _make_sc_gather_scatter(128)


@functools.partial(
    pl.kernel,
    out_type=(jax.ShapeDtypeStruct((NC, N_PAD), jnp.float32),
              jax.ShapeDtypeStruct((NC, N_PAD, H), jnp.float32)),
    mesh=_mesh,
    compiler_params=_sc_params,
    scratch_types=[
        pltpu.VMEM((MC, B), jnp.int32),
        pltpu.VMEM((MC, B), jnp.int32),
        pltpu.VMEM((B, H), jnp.float32),
        pltpu.VMEM((B, H), jnp.float32),
        pltpu.VMEM((B, 2 * H), jnp.float32),
        pltpu.VMEM((B, 2 * H), jnp.float32),
        pltpu.VMEM((B, H), jnp.float32),
        pltpu.VMEM((B, H), jnp.float32),
        pltpu.VMEM((B,), jnp.float32),
        pltpu.VMEM((B,), jnp.float32),
        pltpu.VMEM((B * 16,), jnp.float32),
        pltpu.VMEM((ROWS_PER_TILE,), jnp.float32),
        pltpu.VMEM_SHARED((N_PAD,), jnp.float32),
        pltpu.VMEM_SHARED((N_PAD, H), jnp.float32),
        pltpu.VMEM_SHARED((N, H), jnp.float32),
        [pltpu.SemaphoreType.DMA] * 2,
        [pltpu.SemaphoreType.DMA] * 2,
        [pltpu.SemaphoreType.DMA] * 2,
        [pltpu.SemaphoreType.DMA] * 2,
    ],
)
def _sc_attention(e_src_hbm, e_dst_hbm, q_hbm, kv_hbm,
                  den_out, agg_out,
                  srcb, dstb, qb0, qb1, kvb0, kvb1, sb0, sb1, exb0, exb1,
                  tmat, zstripe, den_acc, agg_acc, q_tbl,
                  semq, semkv, semA, semD):
    wid, cid, sid = _worker_id()
    qbs, kvbs, sbs, exbs = (qb0, qb1), (kvb0, kvb1), (sb0, sb1), (exb0, exb1)
    _fill_zeros(sb0)
    _fill_zeros(zstripe)
    _zero_stripe_2d(agg_acc, sb0, sid)
    _zero_stripe_1d(den_acc, zstripe, sid)
    _q_rows = N // NS
    pltpu.sync_copy(q_hbm.at[pl.ds(sid * _q_rows, _q_rows)],
                    q_tbl.at[pl.ds(sid * _q_rows, _q_rows)])
    plsc.subcore_barrier()

    lane = lax.iota(jnp.int32, 16)

    def gstart(j, p):
        pltpu.async_copy(q_tbl.at[dstb.at[j]], qbs[p], semq[p])
        pltpu.async_copy(kv_hbm.at[srcb.at[j]], kvbs[p], semkv[p])

    def gwait(p):
        pltpu.make_async_copy(q_tbl.at[dstb.at[0]], qbs[p], semq[p]).wait()
        pltpu.make_async_copy(kv_hbm.at[srcb.at[0]], kvbs[p], semkv[p]).wait()

    def swait(p):
        pltpu.make_async_copy(sbs[p], agg_acc.at[dstb.at[0]], semA[p]).wait()
        pltpu.make_async_copy(exbs[p], den_acc.at[dstb.at[0]], semD[p]).wait()

    def compute_and_scatter(j, p):
        qb, kvb, sb, exb = qbs[p], kvbs[p], sbs[p], exbs[p]

        @plsc.parallel_loop(0, B // 16, unroll=4)
        def group(g):
            base = g * 16
            tbase = g * 256
            for e in range(16):
                t0 = qb[base + e, pl.ds(0, 16)] * kvb[base + e, pl.ds(0, 16)]
                t1 = qb[base + e, pl.ds(16, 16)] * kvb[base + e, pl.ds(16, 16)]
                t2 = qb[base + e, pl.ds(32, 16)] * kvb[base + e, pl.ds(32, 16)]
                t3 = qb[base + e, pl.ds(48, 16)] * kvb[base + e, pl.ds(48, 16)]
                plsc.store_scatter(tmat, [tbase + lane * 16 + e], (t0 + t1) + (t2 + t3))
            alpha = jnp.zeros((16,), jnp.float32)
            for r in range(16):
                alpha = alpha + tmat[pl.ds(tbase + r * 16, 16)]
            ex = jnp.exp(alpha)
            exb[pl.ds(base, 16)] = ex
            for e in range(16):
                s = ex[e]
                vv0 = kvb[base + e, pl.ds(H, 16)]
                vv1 = kvb[base + e, pl.ds(H + 16, 16)]
                vv2 = kvb[base + e, pl.ds(H + 32, 16)]
                vv3 = kvb[base + e, pl.ds(H + 48, 16)]
                sb[base + e, pl.ds(0, 16)] = vv0 * s
                sb[base + e, pl.ds(16, 16)] = vv1 * s
                sb[base + e, pl.ds(32, 16)] = vv2 * s
                sb[base + e, pl.ds(48, 16)] = vv3 * s
        pltpu.async_copy(sb, agg_acc.at[dstb.at[j]], semA[p], add=True)
        pltpu.async_copy(exb, den_acc.at[dstb.at[j]], semD[p], add=True)

    def chunk(c, carry):
        pltpu.sync_copy(e_src_hbm.at[wid, pl.ds(c * MC, MC)], srcb)
        pltpu.sync_copy(e_dst_hbm.at[wid, pl.ds(c * MC, MC)], dstb)
        gstart(0, 0)
        gstart(1, 1)

        def body(i, carry2):
            j0 = 2 * i
            for p in range(2):
                j = j0 + p

                @pl.when(j < MC)
                def _(j=j, p=p):
                    gwait(p)

                    @pl.when(j >= 2)
                    def _():
                        swait(p)
                    compute_and_scatter(j, p)

                    @pl.when(j + 2 < MC)
                    def _():
                        gstart(j + 2, p)
            return carry2

        lax.fori_loop(0, (MC + 1) // 2, body, None)
        swait(0)
        swait(1)
        return carry

    lax.fori_loop(0, CH, chunk, None)
    plsc.subcore_barrier()
    _writeback_1d(den_acc, den_out, cid, sid)
    _writeback_2d(agg_acc, agg_out, cid, sid)


def _tc1_body(x_ref, w1_ref, degp_ref, dinv_ref, xw1s_ref):
    deg = degp_ref[0, :N] + degp_ref[1, :N] + 1.0
    dinv = lax.rsqrt(deg)
    dinv_ref[...] = dinv
    xw1 = jnp.dot(x_ref[...], w1_ref[...], preferred_element_type=jnp.float32)
    xw1s_ref[...] = xw1 * dinv


def _tc2_body(g1p_ref, xw1s_ref, dinv_ref, b1_ref,
              wq_ref, bq_ref, wk_ref, bk_ref, wv_ref, bv_ref, ws_ref, bs_ref,
              q_ref, kv_ref, skip_ref):
    ssum = g1p_ref[0, :N] + g1p_ref[1, :N] + xw1s_ref[...]
    x1 = jnp.maximum(ssum * dinv_ref[...] + b1_ref[...], 0.0)
    q_ref[...] = (jnp.dot(x1, wq_ref[...], preferred_element_type=jnp.float32)
                  + bq_ref[...]) * jnp.float32(0.125)
    k = jnp.dot(x1, wk_ref[...], preferred_element_type=jnp.float32) + bk_ref[...]
    v = jnp.dot(x1, wv_ref[...], preferred_element_type=jnp.float32) + bv_ref[...]
    kv_ref[...] = jnp.concatenate([k, v], axis=1)
    skip_ref[...] = jnp.dot(x1, ws_ref[...], preferred_element_type=jnp.float32) + bs_ref[...]


def _tc3_body(aggp_ref, denp_ref, skip_ref, dinv_ref, w2_ref, xw2s_ref):
    den = denp_ref[0, :N] + denp_ref[1, :N]
    agg = aggp_ref[0, :N] + aggp_ref[1, :N]
    x2 = agg / (den + 1e-16) + skip_ref[...]
    xw2 = jnp.dot(x2, w2_ref[...], preferred_element_type=jnp.float32)
    xw2s_ref[...] = xw2 * dinv_ref[...]


def _tc4_body(g2p_ref, xw2s_ref, dinv_ref, b2_ref, out_ref):
    ssum = g2p_ref[0, :N] + g2p_ref[1, :N] + xw2s_ref[...]
    out_ref[...] = ssum * dinv_ref[...] + b2_ref[...]


def kernel(x, edge_index, W1, b1, Wq, bq, Wk, bk, Wv, bv, Ws, bs, W2, b2):
    e_src = edge_index[0].reshape(NW, M, B)
    e_dst = edge_index[1].reshape(NW, M, B)

    degp = _sc_degree(e_dst)

    dinv, xw1s = pl.pallas_call(
        _tc1_body,
        out_shape=(jax.ShapeDtypeStruct((N, 1), jnp.float32),
                   jax.ShapeDtypeStruct((N, H), jnp.float32)),
    )(x, W1, degp.reshape(NC, N_PAD, 1))

    g1p = _sc_gcn_gather_scatter_64(e_src, e_dst, xw1s)

    q, kv, skip = pl.pallas_call(
        _tc2_body,
        out_shape=(jax.ShapeDtypeStruct((N, H), jnp.float32),
                   jax.ShapeDtypeStruct((N, 2 * H), jnp.float32),
                   jax.ShapeDtypeStruct((N, H), jnp.float32)),
    )(g1p, xw1s, dinv, b1.reshape(1, H),
      Wq, bq.reshape(1, H), Wk, bk.reshape(1, H),
      Wv, bv.reshape(1, H), Ws, bs.reshape(1, H))

    denp, aggp = _sc_attention(e_src, e_dst, q, kv)

    xw2s = pl.pallas_call(
        _tc3_body,
        out_shape=jax.ShapeDtypeStruct((N, D_OUT), jnp.float32),
    )(aggp, denp.reshape(NC, N_PAD, 1), skip, dinv, W2)

    g2p = _sc_gcn_gather_scatter_128(e_src, e_dst, xw2s)

    x3 = pl.pallas_call(
        _tc4_body,
        out_shape=jax.ShapeDtypeStruct((N, D_OUT), jnp.float32),
    )(g2p, xw2s, dinv, b2.reshape(1, D_OUT))

    return x3


# speedup vs baseline: 1.2899x; 1.2899x over previous
import functools

import jax
import jax.numpy as jnp
from jax import lax
from jax.experimental import pallas as pl
from jax.experimental.pallas import tpu as pltpu
from jax.experimental.pallas import tpu_sc as plsc

N = 10000
E = 320000
D_IN = 128
H = 64
D_OUT = 128

NC = 2
NS = 16
NW = NC * NS
T = E // NW
B = 80
M = T // B
MC = 25
CH = M // MC
N_PAD = 10240
ROWS_PER_TILE = N_PAD // NS

_mesh = plsc.VectorSubcoreMesh(
    core_axis_name="c", subcore_axis_name="s", num_cores=NC, num_subcores=NS)
_sc_params = pltpu.CompilerParams(use_tc_tiling_on_sc=False, needs_layout_passes=False)


def _worker_id():
    cid = lax.axis_index("c")
    sid = lax.axis_index("s")
    return cid * NS + sid, cid, sid


def _load_edge_block(e3d_hbm, buf, wid):
    pltpu.sync_copy(e3d_hbm.at[wid], buf)


def _fill_zeros(zbuf):
    z = jnp.zeros((16,), jnp.float32)
    if zbuf.ndim == 2:
        for r in range(zbuf.shape[0]):
            for c in range(0, zbuf.shape[1], 16):
                zbuf[r, pl.ds(c, 16)] = z
    else:
        for c in range(0, zbuf.shape[0], 16):
            zbuf[pl.ds(c, 16)] = z


def _zero_stripe_2d(acc, zblk, sid):
    zr = zblk.shape[0]
    base = sid * ROWS_PER_TILE
    for i in range(ROWS_PER_TILE // zr):
        pltpu.sync_copy(zblk, acc.at[pl.ds(base + i * zr, zr)])


def _zero_stripe_1d(acc, zstripe, sid):
    pltpu.sync_copy(zstripe, acc.at[pl.ds(sid * ROWS_PER_TILE, ROWS_PER_TILE)])


def _writeback_2d(acc, out_hbm, cid, sid):
    base = sid * ROWS_PER_TILE
    pltpu.sync_copy(acc.at[pl.ds(base, ROWS_PER_TILE)],
                    out_hbm.at[cid, pl.ds(base, ROWS_PER_TILE)])


def _writeback_1d(acc, out_hbm, cid, sid):
    base = sid * ROWS_PER_TILE
    pltpu.sync_copy(acc.at[pl.ds(base, ROWS_PER_TILE)],
                    out_hbm.at[cid, pl.ds(base, ROWS_PER_TILE)])


@functools.partial(
    pl.kernel,
    out_type=jax.ShapeDtypeStruct((NC, N_PAD), jnp.float32),
    mesh=_mesh,
    compiler_params=_sc_params,
    scratch_types=[
        pltpu.VMEM((M, B), jnp.int32),
        pltpu.VMEM((B,), jnp.float32),
        pltpu.VMEM((ROWS_PER_TILE,), jnp.float32),
        pltpu.VMEM_SHARED((N_PAD,), jnp.float32),
    ],
)
def _sc_degree(e_dst_hbm, out_hbm, dstb, ones, zstripe, acc):
    wid, cid, sid = _worker_id()
    _fill_zeros(zstripe)
    one = jnp.ones((16,), jnp.float32)
    for c in range(0, B, 16):
        ones[pl.ds(c, 16)] = one
    _zero_stripe_1d(acc, zstripe, sid)
    _load_edge_block(e_dst_hbm, dstb, wid)
    plsc.subcore_barrier()

    def body(j, carry):
        pltpu.sync_copy(ones, acc.at[dstb.at[j]], add=True)
        return carry

    lax.fori_loop(0, M, body, None)
    plsc.subcore_barrier()
    _writeback_1d(acc, out_hbm, cid, sid)


def _make_sc_gather_scatter(D, stage_table=False):
    NB = 4

    scratch = [
        pltpu.VMEM((MC, B), jnp.int32),
        pltpu.VMEM((MC, B), jnp.int32),
        pltpu.VMEM((B, D), jnp.float32),
        pltpu.VMEM((B, D), jnp.float32),
        pltpu.VMEM((B, D), jnp.float32),
        pltpu.VMEM((B, D), jnp.float32),
        pltpu.VMEM_SHARED((N_PAD, D), jnp.float32),
        [pltpu.SemaphoreType.DMA] * 4,
        [pltpu.SemaphoreType.DMA] * 4,
    ]
    if stage_table:
        scratch.append(pltpu.VMEM_SHARED((N, D), jnp.float32))

    @functools.partial(
        pl.kernel,
        out_type=jax.ShapeDtypeStruct((NC, N_PAD, D), jnp.float32),
        mesh=_mesh,
        compiler_params=_sc_params,
        scratch_types=scratch,
    )
    def _sc_gs(e_src_hbm, e_dst_hbm, table_hbm, out_hbm,
               srcb, dstb, b0, b1, b2, b3, acc, gsems, ssems, *maybe_tbl):
        wid, cid, sid = _worker_id()
        bufs = (b0, b1, b2, b3)
        _fill_zeros(b0)
        _zero_stripe_2d(acc, b0, sid)
        if stage_table:
            tbl = maybe_tbl[0]
            rows = N // NS
            pltpu.sync_copy(table_hbm.at[pl.ds(sid * rows, rows)],
                            tbl.at[pl.ds(sid * rows, rows)])
            table_src = tbl
        else:
            table_src = table_hbm
        plsc.subcore_barrier()

        def gstart(j, p):
            pltpu.async_copy(table_src.at[srcb.at[j]], bufs[p], gsems[p])

        def gwait(p):
            pltpu.make_async_copy(table_src.at[srcb.at[0]], bufs[p], gsems[p]).wait()

        def sstart(j, p):
            pltpu.async_copy(bufs[p], acc.at[dstb.at[j]], ssems[p], add=True)

        def swait(p):
            pltpu.make_async_copy(bufs[p], acc.at[dstb.at[0]], ssems[p]).wait()

        def chunk(c, carry):
            pltpu.sync_copy(e_src_hbm.at[wid, pl.ds(c * MC, MC)], srcb)
            pltpu.sync_copy(e_dst_hbm.at[wid, pl.ds(c * MC, MC)], dstb)
            gstart(0, 0)
            gstart(1, 1)

            def body(i, carry2):
                j0 = 4 * i
                for p in range(NB):
                    j = j0 + p

                    @pl.when(j < MC)
                    def _(j=j, p=p):
                        pn = (p + 2) % NB

                        @pl.when(j + 2 < MC)
                        def _():
                            @pl.when(j >= 2)
                            def _():
                                swait(pn)
                            gstart(j + 2, pn)

                        gwait(p)
                        sstart(j, p)
                return carry2

            lax.fori_loop(0, (MC + NB - 1) // NB, body, None)
            for p in range(NB):
                swait(p)
            return carry

        lax.fori_loop(0, CH, chunk, None)
        plsc.subcore_barrier()
        _writeback_2d(acc, out_hbm, cid, sid)

    return _sc_gs


_sc_gcn_gather_scatter_64 = _make_sc_gather_scatter(64, stage_table=True)
_sc_gcn_gather_scatter_128 = _make_sc_gather_scatter(128)


@functools.partial(
    pl.kernel,
    out_type=(jax.ShapeDtypeStruct((NC, N_PAD), jnp.float32),
              jax.ShapeDtypeStruct((NC, N_PAD, H), jnp.float32)),
    mesh=_mesh,
    compiler_params=_sc_params,
    scratch_types=[
        pltpu.VMEM((MC, B), jnp.int32),
        pltpu.VMEM((MC, B), jnp.int32),
        pltpu.VMEM((B, H), jnp.float32),
        pltpu.VMEM((B, H), jnp.float32),
        pltpu.VMEM((B, 2 * H), jnp.float32),
        pltpu.VMEM((B, 2 * H), jnp.float32),
        pltpu.VMEM((B, H), jnp.float32),
        pltpu.VMEM((B, H), jnp.float32),
        pltpu.VMEM((B,), jnp.float32),
        pltpu.VMEM((B,), jnp.float32),
        pltpu.VMEM((B * 16,), jnp.float32),
        pltpu.VMEM((ROWS_PER_TILE,), jnp.float32),
        pltpu.VMEM_SHARED((N_PAD,), jnp.float32),
        pltpu.VMEM_SHARED((N_PAD, H), jnp.float32),
        [pltpu.SemaphoreType.DMA] * 2,
        [pltpu.SemaphoreType.DMA] * 2,
        [pltpu.SemaphoreType.DMA] * 2,
        [pltpu.SemaphoreType.DMA] * 2,
    ],
)
def _sc_attention(e_src_hbm, e_dst_hbm, q_hbm, kv_hbm,
                  den_out, agg_out,
                  srcb, dstb, qb0, qb1, kvb0, kvb1, sb0, sb1, exb0, exb1,
                  tmat, zstripe, den_acc, agg_acc,
                  semq, semkv, semA, semD):
    wid, cid, sid = _worker_id()
    qbs, kvbs, sbs, exbs = (qb0, qb1), (kvb0, kvb1), (sb0, sb1), (exb0, exb1)
    _fill_zeros(sb0)
    _fill_zeros(zstripe)
    _zero_stripe_2d(agg_acc, sb0, sid)
    _zero_stripe_1d(den_acc, zstripe, sid)
    plsc.subcore_barrier()

    lane = lax.iota(jnp.int32, 16)

    def gstart(j, p):
        pltpu.async_copy(q_hbm.at[dstb.at[j]], qbs[p], semq[p])
        pltpu.async_copy(kv_hbm.at[srcb.at[j]], kvbs[p], semkv[p])

    def gwait(p):
        pltpu.make_async_copy(q_hbm.at[dstb.at[0]], qbs[p], semq[p]).wait()
        pltpu.make_async_copy(kv_hbm.at[srcb.at[0]], kvbs[p], semkv[p]).wait()

    def swait(p):
        pltpu.make_async_copy(sbs[p], agg_acc.at[dstb.at[0]], semA[p]).wait()
        pltpu.make_async_copy(exbs[p], den_acc.at[dstb.at[0]], semD[p]).wait()

    def compute_and_scatter(j, p):
        qb, kvb, sb, exb = qbs[p], kvbs[p], sbs[p], exbs[p]

        @plsc.parallel_loop(0, B // 16)
        def group(g):
            base = g * 16
            tbase = g * 256

            @plsc.parallel_loop(0, 16, unroll=8)
            def dot_e(e):
                idx = base + e
                t0 = qb[idx, pl.ds(0, 16)] * kvb[idx, pl.ds(0, 16)]
                t1 = qb[idx, pl.ds(16, 16)] * kvb[idx, pl.ds(16, 16)]
                t2 = qb[idx, pl.ds(32, 16)] * kvb[idx, pl.ds(32, 16)]
                t3 = qb[idx, pl.ds(48, 16)] * kvb[idx, pl.ds(48, 16)]
                plsc.store_scatter(tmat, [tbase + lane * 16 + e], (t0 + t1) + (t2 + t3))

            alpha = jnp.zeros((16,), jnp.float32)
            for r in range(16):
                alpha = alpha + tmat[pl.ds(tbase + r * 16, 16)]
            ex = jnp.exp(alpha)
            exb[pl.ds(base, 16)] = ex

            @plsc.parallel_loop(0, 16, unroll=8)
            def scale_e(e):
                idx = base + e
                s = plsc.load_gather(exb, [jnp.zeros((16,), jnp.int32) + idx])
                sb[idx, pl.ds(0, 16)] = kvb[idx, pl.ds(H, 16)] * s
                sb[idx, pl.ds(16, 16)] = kvb[idx, pl.ds(H + 16, 16)] * s
                sb[idx, pl.ds(32, 16)] = kvb[idx, pl.ds(H + 32, 16)] * s
                sb[idx, pl.ds(48, 16)] = kvb[idx, pl.ds(H + 48, 16)] * s
        pltpu.async_copy(sb, agg_acc.at[dstb.at[j]], semA[p], add=True)
        pltpu.async_copy(exb, den_acc.at[dstb.at[j]], semD[p], add=True)

    def chunk(c, carry):
        pltpu.sync_copy(e_src_hbm.at[wid, pl.ds(c * MC, MC)], srcb)
        pltpu.sync_copy(e_dst_hbm.at[wid, pl.ds(c * MC, MC)], dstb)
        gstart(0, 0)
        gstart(1, 1)

        def body(i, carry2):
            j0 = 2 * i
            for p in range(2):
                j = j0 + p

                @pl.when(j < MC)
                def _(j=j, p=p):
                    gwait(p)

                    @pl.when(j >= 2)
                    def _():
                        swait(p)
                    compute_and_scatter(j, p)

                    @pl.when(j + 2 < MC)
                    def _():
                        gstart(j + 2, p)
            return carry2

        lax.fori_loop(0, (MC + 1) // 2, body, None)
        swait(0)
        swait(1)
        return carry

    lax.fori_loop(0, CH, chunk, None)
    plsc.subcore_barrier()
    _writeback_1d(den_acc, den_out, cid, sid)
    _writeback_2d(agg_acc, agg_out, cid, sid)


def _tc1_body(x_ref, w1_ref, degp_ref, dinv_ref, xw1s_ref):
    deg = degp_ref[0, :N] + degp_ref[1, :N] + 1.0
    dinv = lax.rsqrt(deg)
    dinv_ref[...] = dinv
    xw1 = jnp.dot(x_ref[...], w1_ref[...], preferred_element_type=jnp.float32)
    xw1s_ref[...] = xw1 * dinv


def _tc2_body(g1p_ref, xw1s_ref, dinv_ref, b1_ref,
              wq_ref, bq_ref, wk_ref, bk_ref, wv_ref, bv_ref, ws_ref, bs_ref,
              q_ref, kv_ref, skip_ref):
    ssum = g1p_ref[0, :N] + g1p_ref[1, :N] + xw1s_ref[...]
    x1 = jnp.maximum(ssum * dinv_ref[...] + b1_ref[...], 0.0)
    q_ref[...] = (jnp.dot(x1, wq_ref[...], preferred_element_type=jnp.float32)
                  + bq_ref[...]) * jnp.float32(0.125)
    k = jnp.dot(x1, wk_ref[...], preferred_element_type=jnp.float32) + bk_ref[...]
    v = jnp.dot(x1, wv_ref[...], preferred_element_type=jnp.float32) + bv_ref[...]
    kv_ref[...] = jnp.concatenate([k, v], axis=1)
    skip_ref[...] = jnp.dot(x1, ws_ref[...], preferred_element_type=jnp.float32) + bs_ref[...]


def _tc3_body(aggp_ref, denp_ref, skip_ref, dinv_ref, w2_ref, xw2s_ref):
    den = denp_ref[0, :N] + denp_ref[1, :N]
    agg = aggp_ref[0, :N] + aggp_ref[1, :N]
    x2 = agg / (den + 1e-16) + skip_ref[...]
    xw2 = jnp.dot(x2, w2_ref[...], preferred_element_type=jnp.float32)
    xw2s_ref[...] = xw2 * dinv_ref[...]


def _tc4_body(g2p_ref, xw2s_ref, dinv_ref, b2_ref, out_ref):
    ssum = g2p_ref[0, :N] + g2p_ref[1, :N] + xw2s_ref[...]
    out_ref[...] = ssum * dinv_ref[...] + b2_ref[...]


def kernel(x, edge_index, W1, b1, Wq, bq, Wk, bk, Wv, bv, Ws, bs, W2, b2):
    e_src = edge_index[0].reshape(NW, M, B)
    e_dst = edge_index[1].reshape(NW, M, B)

    degp = _sc_degree(e_dst)

    dinv, xw1s = pl.pallas_call(
        _tc1_body,
        out_shape=(jax.ShapeDtypeStruct((N, 1), jnp.float32),
                   jax.ShapeDtypeStruct((N, H), jnp.float32)),
    )(x, W1, degp.reshape(NC, N_PAD, 1))

    g1p = _sc_gcn_gather_scatter_64(e_src, e_dst, xw1s)

    q, kv, skip = pl.pallas_call(
        _tc2_body,
        out_shape=(jax.ShapeDtypeStruct((N, H), jnp.float32),
                   jax.ShapeDtypeStruct((N, 2 * H), jnp.float32),
                   jax.ShapeDtypeStruct((N, H), jnp.float32)),
    )(g1p, xw1s, dinv, b1.reshape(1, H),
      Wq, bq.reshape(1, H), Wk, bk.reshape(1, H),
      Wv, bv.reshape(1, H), Ws, bs.reshape(1, H))

    denp, aggp = _sc_attention(e_src, e_dst, q, kv)

    xw2s = pl.pallas_call(
        _tc3_body,
        out_shape=jax.ShapeDtypeStruct((N, D_OUT), jnp.float32),
    )(aggp, denp.reshape(NC, N_PAD, 1), skip, dinv, W2)

    g2p = _sc_gcn_gather_scatter_128(e_src, e_dst, xw2s)

    x3 = pl.pallas_call(
        _tc4_body,
        out_shape=jax.ShapeDtypeStruct((N, D_OUT), jnp.float32),
    )(g2p, xw2s, dinv, b2.reshape(1, D_OUT))

    return x3
